# baseline (speedup 1.0000x reference)
import jax
import jax.numpy as jnp
from jax.experimental import pallas as pl

N = 8192
D = 32
C = 13
NSAMPLE = 16
TEMP = 0.1
EPS = 1e-7
BQ = 256
OHW = 128

_INF = 3.0e38


def _body(pb_ref, paT_ref, fb_ref, faT_ref, labc_ref, ohp_ref,
          ls_ref, ms_ref):
    i = pl.program_id(0)

    pb = pb_ref[...]
    paT = paT_ref[...]

    mm2 = jnp.dot((pb * -2.0).astype(jnp.bfloat16), paT.astype(jnp.bfloat16),
                  preferred_element_type=jnp.float32)
    d2 = (jnp.sum(pb * pb, axis=1, keepdims=True)
          + (jnp.sum(paT * paT, axis=0, keepdims=True) + mm2))

    nch = 32
    w = N // nch
    chunks = [d2[:, c * w:(c + 1) * w] for c in range(nch)]
    m1 = chunks[0]
    for c in chunks[1:]:
        m1 = jnp.minimum(m1, c)
    m2 = jnp.full((BQ, w), _INF, jnp.float32)
    for c in chunks:
        m2 = jnp.minimum(m2, jnp.where(c > m1, c, _INF))
    red = jnp.concatenate([m1, m2], axis=1)
    t0 = jnp.min(m1, axis=1, keepdims=True)
    t = t0
    for _ in range(NSAMPLE):
        t = jnp.min(jnp.where(red > t, red, _INF), axis=1, keepdims=True)
    nmask = jnp.logical_and(d2 <= t, d2 > t0)

    fb = fb_ref[...]
    faT = faT_ref[...]
    log2e = 1.4426950408889634
    s = (log2e / TEMP) ** 2
    fbs = fb * (-2.0 * s)
    fb_hi = fbs.astype(jnp.bfloat16)
    fb_lo = (fbs - fb_hi.astype(jnp.float32)).astype(jnp.bfloat16)
    fa_hi = faT.astype(jnp.bfloat16)
    fa_lo = (faT - fa_hi.astype(jnp.float32)).astype(jnp.bfloat16)
    fmm = (jnp.dot(fb_hi, fa_hi, preferred_element_type=jnp.float32)
           + jnp.dot(fb_hi, fa_lo, preferred_element_type=jnp.float32)
           + jnp.dot(fb_lo, fa_hi, preferred_element_type=jnp.float32))
    fd2 = (s * jnp.sum(fb * fb, axis=1, keepdims=True)
           + (s * jnp.sum(faT * faT, axis=0, keepdims=True) + fmm))
    x = jnp.where(nmask, jnp.maximum(fd2, 0.0) + EPS * s, 1.0e12)
    xmin = jnp.min(x, axis=1, keepdims=True)
    dist = x * jax.lax.rsqrt(x)
    dmin = jnp.sqrt(xmin)
    e = jnp.exp2(dmin - dist)

    e16 = e.astype(jnp.bfloat16)
    nm16 = jnp.where(nmask, 1.0, 0.0).astype(jnp.bfloat16)
    ohp = ohp_ref[...]
    esum = jnp.dot(e16, ohp, preferred_element_type=jnp.float32)
    csum = jnp.dot(nm16, ohp, preferred_element_type=jnp.float32)
    cls = jax.lax.broadcasted_iota(jnp.int32, (BQ, OHW), 1)
    ohb = labc_ref[...] == cls
    pos = jnp.sum(jnp.where(ohb, esum, 0.0), axis=1, keepdims=True)
    pcnt = jnp.sum(jnp.where(ohb, csum, 0.0), axis=1, keepdims=True)
    neg = esum[:, C:C + 1]
    pm = jnp.logical_and(pcnt > 0.5, pcnt < NSAMPLE - 0.5).astype(jnp.float32)
    lpp = -jnp.log(pos / neg + EPS)

    pls = jnp.sum(lpp * pm, axis=0, keepdims=True)
    pms = jnp.sum(pm, axis=0, keepdims=True)

    @pl.when(i == 0)
    def _():
        ls_ref[...] = pls
        ms_ref[...] = pms

    @pl.when(i > 0)
    def _():
        ls_ref[...] += pls
        ms_ref[...] += pms


def kernel(p, features, labels):
    p = p.astype(jnp.float32)
    features = features.astype(jnp.float32)
    pp = jnp.pad(p, ((0, 0), (0, 5)))
    paT = pp.T
    faT = features.T
    labc = labels.astype(jnp.int32).reshape(N, 1)
    oh = jax.nn.one_hot(labels, C, dtype=jnp.float32)
    ohp = jnp.concatenate(
        [oh, jnp.ones((N, 1), jnp.float32),
         jnp.zeros((N, OHW - C - 1), jnp.float32)],
        axis=1).astype(jnp.bfloat16)

    ls, ms = pl.pallas_call(
        _body,
        grid=(N // BQ,),
        in_specs=[
            pl.BlockSpec((BQ, 8), lambda i: (i, 0)),
            pl.BlockSpec((8, N), lambda i: (0, 0)),
            pl.BlockSpec((BQ, D), lambda i: (i, 0)),
            pl.BlockSpec((D, N), lambda i: (0, 0)),
            pl.BlockSpec((BQ, 1), lambda i: (i, 0)),
            pl.BlockSpec((N, OHW), lambda i: (0, 0)),
        ],
        out_specs=[
            pl.BlockSpec((1, 1), lambda i: (0, 0)),
            pl.BlockSpec((1, 1), lambda i: (0, 0)),
        ],
        out_shape=[
            jax.ShapeDtypeStruct((1, 1), jnp.float32),
            jax.ShapeDtypeStruct((1, 1), jnp.float32),
        ],
        interpret=_INTERPRET,
    )(pp, paT, features, faT, labc, ohp)

    return (ls[0, 0] / jnp.maximum(ms[0, 0], 1.0)).astype(jnp.float32)


_INTERPRET = False


# speedup vs baseline: 1.0243x; 1.0243x over previous
import jax
import jax.numpy as jnp
from jax.experimental import pallas as pl

N = 8192
D = 32
C = 13
NSAMPLE = 16
TEMP = 0.1
EPS = 1e-7
BQ = 256
OHW = 128

_INF = 3.0e38


def _body(pb_ref, paT_ref, fb_ref, faT_ref, labc_ref, labr_ref,
          ls_ref, ms_ref):
    i = pl.program_id(0)

    pb = pb_ref[...]
    paT = paT_ref[...]

    mm2 = jnp.dot((pb * -2.0).astype(jnp.bfloat16), paT.astype(jnp.bfloat16),
                  preferred_element_type=jnp.float32)
    d2 = (jnp.sum(pb * pb, axis=1, keepdims=True)
          + (jnp.sum(paT * paT, axis=0, keepdims=True) + mm2))

    nch = 32
    w = N // nch
    chunks = [d2[:, c * w:(c + 1) * w] for c in range(nch)]
    m1 = chunks[0]
    for c in chunks[1:]:
        m1 = jnp.minimum(m1, c)
    m2 = jnp.full((BQ, w), _INF, jnp.float32)
    for c in chunks:
        m2 = jnp.minimum(m2, jnp.where(c > m1, c, _INF))
    red = jnp.concatenate([m1, m2], axis=1)
    t0 = jnp.min(m1, axis=1, keepdims=True)
    t = t0
    for _ in range(NSAMPLE):
        t = jnp.min(jnp.where(red > t, red, _INF), axis=1, keepdims=True)
    nmask = jnp.logical_and(d2 <= t, d2 > t0)

    fb = fb_ref[...]
    faT = faT_ref[...]
    log2e = 1.4426950408889634
    s = (log2e / TEMP) ** 2
    fbs = fb * (-2.0 * s)
    fb_hi = fbs.astype(jnp.bfloat16)
    fb_lo = (fbs - fb_hi.astype(jnp.float32)).astype(jnp.bfloat16)
    fa_hi = faT.astype(jnp.bfloat16)
    fa_lo = (faT - fa_hi.astype(jnp.float32)).astype(jnp.bfloat16)
    fmm = (jnp.dot(fb_hi, fa_hi, preferred_element_type=jnp.float32)
           + jnp.dot(fb_hi, fa_lo, preferred_element_type=jnp.float32)
           + jnp.dot(fb_lo, fa_hi, preferred_element_type=jnp.float32))
    fd2 = (s * jnp.sum(fb * fb, axis=1, keepdims=True)
           + (s * jnp.sum(faT * faT, axis=0, keepdims=True) + fmm))
    x = jnp.where(nmask, jnp.maximum(fd2, 0.0) + EPS * s, 1.0e12)
    xmin = jnp.min(x, axis=1, keepdims=True)
    dist = x * jax.lax.rsqrt(x)
    dmin = jnp.sqrt(xmin)
    e = jnp.exp2(dmin - dist)

    eq = labc_ref[...] == labr_ref[...]
    pos = jnp.sum(jnp.where(eq, e, 0.0), axis=1, keepdims=True)
    neg = jnp.sum(e, axis=1, keepdims=True)
    pcnt = jnp.sum(jnp.where(jnp.logical_and(eq, nmask), 1.0, 0.0),
                   axis=1, keepdims=True)
    pm = jnp.logical_and(pcnt > 0.5, pcnt < NSAMPLE - 0.5).astype(jnp.float32)
    lpp = -jnp.log(pos / neg + EPS)

    pls = jnp.sum(lpp * pm, axis=0, keepdims=True)
    pms = jnp.sum(pm, axis=0, keepdims=True)

    @pl.when(i == 0)
    def _():
        ls_ref[...] = pls
        ms_ref[...] = pms

    @pl.when(i > 0)
    def _():
        ls_ref[...] += pls
        ms_ref[...] += pms


def kernel(p, features, labels):
    p = p.astype(jnp.float32)
    features = features.astype(jnp.float32)
    pp = jnp.pad(p, ((0, 0), (0, 5)))
    paT = pp.T
    faT = features.T
    labi = labels.astype(jnp.int32)
    labc = labi.reshape(N, 1)
    labr = labi.reshape(1, N)

    ls, ms = pl.pallas_call(
        _body,
        grid=(N // BQ,),
        in_specs=[
            pl.BlockSpec((BQ, 8), lambda i: (i, 0)),
            pl.BlockSpec((8, N), lambda i: (0, 0)),
            pl.BlockSpec((BQ, D), lambda i: (i, 0)),
            pl.BlockSpec((D, N), lambda i: (0, 0)),
            pl.BlockSpec((BQ, 1), lambda i: (i, 0)),
            pl.BlockSpec((1, N), lambda i: (0, 0)),
        ],
        out_specs=[
            pl.BlockSpec((1, 1), lambda i: (0, 0)),
            pl.BlockSpec((1, 1), lambda i: (0, 0)),
        ],
        out_shape=[
            jax.ShapeDtypeStruct((1, 1), jnp.float32),
            jax.ShapeDtypeStruct((1, 1), jnp.float32),
        ],
        interpret=_INTERPRET,
    )(pp, paT, features, faT, labc, labr)

    return (ls[0, 0] / jnp.maximum(ms[0, 0], 1.0)).astype(jnp.float32)


_INTERPRET = False


# speedup vs baseline: 1.0635x; 1.0383x over previous
import jax
import jax.numpy as jnp
from jax.experimental import pallas as pl

N = 8192
D = 32
C = 13
NSAMPLE = 16
TEMP = 0.1
EPS = 1e-7
BQ = 256
OHW = 128

_INF = 3.0e38


def _body(pb_ref, paT_ref, fb_ref, faT_ref, labc_ref, labr_ref,
          ls_ref, ms_ref):
    i = pl.program_id(0)

    pb = pb_ref[...]
    paT = paT_ref[...]

    mm2 = jnp.dot((pb * -2.0).astype(jnp.bfloat16), paT.astype(jnp.bfloat16),
                  preferred_element_type=jnp.float32)
    d2 = (jnp.sum(pb * pb, axis=1, keepdims=True)
          + (jnp.sum(paT * paT, axis=0, keepdims=True) + mm2))

    nch = 64
    w = N // nch
    chunks = [d2[:, c * w:(c + 1) * w] for c in range(nch)]
    m1 = chunks[0]
    m2 = jnp.full((BQ, w), _INF, jnp.float32)
    for c in chunks[1:]:
        m2 = jnp.minimum(m2, jnp.maximum(m1, c))
        m1 = jnp.minimum(m1, c)
    red = jnp.concatenate([m1, m2], axis=1)
    t0 = jnp.min(m1, axis=1, keepdims=True)
    t = t0
    for _ in range(NSAMPLE):
        t = jnp.min(jnp.where(red > t, red, _INF), axis=1, keepdims=True)
    nmask = jnp.logical_and(d2 <= t, d2 > t0)

    fb = fb_ref[...]
    faT = faT_ref[...]
    log2e = 1.4426950408889634
    s = (log2e / TEMP) ** 2
    fbs = fb * (-2.0 * s)
    fb_hi = fbs.astype(jnp.bfloat16)
    fb_lo = (fbs - fb_hi.astype(jnp.float32)).astype(jnp.bfloat16)
    fa_hi = faT.astype(jnp.bfloat16)
    fa_lo = (faT - fa_hi.astype(jnp.float32)).astype(jnp.bfloat16)
    fmm = (jnp.dot(fb_hi, fa_hi, preferred_element_type=jnp.float32)
           + jnp.dot(fb_hi, fa_lo, preferred_element_type=jnp.float32)
           + jnp.dot(fb_lo, fa_hi, preferred_element_type=jnp.float32))
    fd2 = (s * jnp.sum(fb * fb, axis=1, keepdims=True)
           + (s * jnp.sum(faT * faT, axis=0, keepdims=True) + fmm))
    x = jnp.where(nmask, jnp.maximum(fd2, 0.0) + EPS * s, 1.0e12)
    xmin = jnp.min(x, axis=1, keepdims=True)
    dist = x * jax.lax.rsqrt(x)
    dmin = jnp.sqrt(xmin)
    e = jnp.exp2(dmin - dist)

    eq = labc_ref[...] == labr_ref[...]
    pos = jnp.sum(jnp.where(eq, e, 0.0), axis=1, keepdims=True)
    neg = jnp.sum(e, axis=1, keepdims=True)
    pcnt = jnp.sum(jnp.where(jnp.logical_and(eq, nmask), 1.0, 0.0),
                   axis=1, keepdims=True)
    pm = jnp.logical_and(pcnt > 0.5, pcnt < NSAMPLE - 0.5).astype(jnp.float32)
    lpp = -jnp.log(pos / neg + EPS)

    pls = jnp.sum(lpp * pm, axis=0, keepdims=True)
    pms = jnp.sum(pm, axis=0, keepdims=True)

    @pl.when(i == 0)
    def _():
        ls_ref[...] = pls
        ms_ref[...] = pms

    @pl.when(i > 0)
    def _():
        ls_ref[...] += pls
        ms_ref[...] += pms


def kernel(p, features, labels):
    p = p.astype(jnp.float32)
    features = features.astype(jnp.float32)
    pp = jnp.pad(p, ((0, 0), (0, 5)))
    paT = pp.T
    faT = features.T
    labi = labels.astype(jnp.int32)
    labc = labi.reshape(N, 1)
    labr = labi.reshape(1, N)

    ls, ms = pl.pallas_call(
        _body,
        grid=(N // BQ,),
        in_specs=[
            pl.BlockSpec((BQ, 8), lambda i: (i, 0)),
            pl.BlockSpec((8, N), lambda i: (0, 0)),
            pl.BlockSpec((BQ, D), lambda i: (i, 0)),
            pl.BlockSpec((D, N), lambda i: (0, 0)),
            pl.BlockSpec((BQ, 1), lambda i: (i, 0)),
            pl.BlockSpec((1, N), lambda i: (0, 0)),
        ],
        out_specs=[
            pl.BlockSpec((1, 1), lambda i: (0, 0)),
            pl.BlockSpec((1, 1), lambda i: (0, 0)),
        ],
        out_shape=[
            jax.ShapeDtypeStruct((1, 1), jnp.float32),
            jax.ShapeDtypeStruct((1, 1), jnp.float32),
        ],
        interpret=_INTERPRET,
    )(pp, paT, features, faT, labc, labr)

    return (ls[0, 0] / jnp.maximum(ms[0, 0], 1.0)).astype(jnp.float32)


_INTERPRET = False
